# baseline (speedup 1.0000x reference)
import functools

import jax
import jax.numpy as jnp
from jax import lax
from jax.experimental import pallas as pl
from jax.experimental.pallas import tpu as pltpu
from jax.experimental.pallas import tpu_sc as plsc

_B, _S, _D, _R, _E = 4, 2048, 2048, 64, 8
_TOPK = 2
_SCALING = 16.0 / _R
_S_TILE = 1024
_NEG = -1e30
_LANES = 16


def _lane_sum(v):
    idx = lax.iota(jnp.int32, _LANES)
    for k in (8, 4, 2, 1):
        v = v + v.at[idx ^ k].get(mode="promise_in_bounds")
    return v[0]


def _sc_logits_body(x_hbm, eof_hbm, route_hbm, out_hbm,
                    eof_v, row_v, route_v, lv):
    c = lax.axis_index("c")
    s = lax.axis_index("s")
    b = c * 2 + s // _E
    e = s % _E

    pltpu.sync_copy(eof_hbm, eof_v)
    ev = eof_v[...]
    fidx = ev.at[jnp.full((_LANES,), b, jnp.int32)].get(
        mode="promise_in_bounds") + b * _S

    pltpu.sync_copy(x_hbm.at[fidx], row_v)
    pltpu.sync_copy(route_hbm.at[pl.ds(e, 1)], route_v)

    rowr = row_v.at[0]
    rtr = route_v.at[0]

    def chunk(k, acc):
        o = k * _LANES
        return acc + rowr[pl.ds(o, _LANES)] * rtr[pl.ds(o, _LANES)]

    acc = lax.fori_loop(0, _D // _LANES, chunk,
                        jnp.zeros((_LANES,), jnp.float32))
    logit = _lane_sum(acc)

    lv[0, 0] = jnp.full((_LANES,), logit, jnp.float32)
    pltpu.sync_copy(lv, out_hbm.at[pl.ds(b, 1), pl.ds(e, 1)])


_sc_logits = functools.partial(
    pl.kernel,
    out_type=jax.ShapeDtypeStruct((_B, _E, _LANES), jnp.float32),
    mesh=plsc.VectorSubcoreMesh(core_axis_name="c", subcore_axis_name="s",
                                num_cores=2, num_subcores=16),
    scratch_types=[
        pltpu.VMEM((_LANES,), jnp.int32),
        pltpu.VMEM((_LANES, _D), jnp.float32),
        pltpu.VMEM((1, _D), jnp.float32),
        pltpu.VMEM((1, 1, _LANES), jnp.float32),
    ],
)(_sc_logits_body)


def _tc_body(lt_ref, x_ref, a_ref, bs_ref, o_ref, cb_ref):
    b = pl.program_id(0)
    s = pl.program_id(1)

    @pl.when(s == 0)
    def _routing():
        iota = lax.broadcasted_iota(jnp.int32, (_B, _E, _LANES), 0)
        lrow = jnp.sum(jnp.where(iota == b, lt_ref[...], 0.0),
                       axis=0)
        logits = [lrow[e, 0] for e in range(_E)]
        m1 = logits[0]
        i1 = jnp.int32(0)
        for e in range(1, _E):
            gt = logits[e] > m1
            i1 = jnp.where(gt, jnp.int32(e), i1)
            m1 = jnp.where(gt, logits[e], m1)
        m2 = jnp.float32(_NEG)
        i2 = jnp.int32(0)
        for e in range(_E):
            gt = (logits[e] > m2) & (i1 != e)
            i2 = jnp.where(gt, jnp.int32(e), i2)
            m2 = jnp.where(gt, logits[e], m2)
        e2 = jnp.exp(m2 - m1)
        denom = 1.0 + e2
        g1 = (_SCALING / denom)
        g2 = (_SCALING * e2 / denom)
        cb = jnp.zeros((_D, _R), dtype=jnp.float32)
        for e in range(_E):
            w_e = jnp.where(i1 == e, g1, 0.0) + jnp.where(i2 == e, g2, 0.0)
            cb = cb + w_e * bs_ref[e]
        cb_ref[...] = cb.astype(jnp.bfloat16)

    xt = x_ref[0].astype(jnp.bfloat16)
    shared = lax.dot_general(
        xt, a_ref[...].astype(jnp.bfloat16), (((1,), (1,)), ((), ())),
        preferred_element_type=jnp.float32)
    out = lax.dot_general(
        shared.astype(jnp.bfloat16), cb_ref[...], (((1,), (1,)), ((), ())),
        preferred_element_type=jnp.float32)
    o_ref[0] = out


@jax.jit
def kernel(x, eof_index, lora_A, lora_route, lora_Bs):
    eof16 = jnp.concatenate(
        [eof_index, jnp.zeros((_LANES - _B,), jnp.int32)])
    logits_tab = _sc_logits(x.reshape(_B * _S, _D), eof16, lora_route)
    grid = (_B, _S // _S_TILE)
    return pl.pallas_call(
        _tc_body,
        grid=grid,
        in_specs=[
            pl.BlockSpec((_B, _E, _LANES), lambda b, s: (0, 0, 0)),
            pl.BlockSpec((1, _S_TILE, _D), lambda b, s: (b, s, 0)),
            pl.BlockSpec((_R, _D), lambda b, s: (0, 0)),
            pl.BlockSpec((_E, _D, _R), lambda b, s: (0, 0, 0)),
        ],
        out_specs=pl.BlockSpec((1, _S_TILE, _D), lambda b, s: (b, s, 0)),
        scratch_shapes=[pltpu.VMEM((_D, _R), jnp.bfloat16)],
        out_shape=jax.ShapeDtypeStruct((_B, _S, _D), jnp.float32),
    )(logits_tab, x, lora_A, lora_Bs)


# speedup vs baseline: 1.0161x; 1.0161x over previous
import functools

import jax
import jax.numpy as jnp
from jax import lax
from jax.experimental import pallas as pl
from jax.experimental.pallas import tpu as pltpu
from jax.experimental.pallas import tpu_sc as plsc

_B, _S, _D, _R, _E = 4, 2048, 2048, 64, 8
_TOPK = 2
_SCALING = 16.0 / _R
_S_TILE = 1024
_NEG = -1e30
_LANES = 16


def _lane_sum(v):
    idx = lax.iota(jnp.int32, _LANES)
    for k in (8, 4, 2, 1):
        v = v + v.at[idx ^ k].get(mode="promise_in_bounds")
    return v[0]


def _sc_logits_body(x_hbm, eof_hbm, route_hbm, out_hbm,
                    eof_v, row_v, route_v, lv):
    c = lax.axis_index("c")
    s = lax.axis_index("s")
    b = c * 2 + s // _E
    e = s % _E

    pltpu.sync_copy(eof_hbm, eof_v)
    ev = eof_v[...]
    fidx = ev.at[jnp.full((_LANES,), b, jnp.int32)].get(
        mode="promise_in_bounds") + b * _S

    pltpu.sync_copy(x_hbm.at[fidx], row_v)
    pltpu.sync_copy(route_hbm.at[pl.ds(e, 1)], route_v)

    rowr = row_v.at[0]
    rtr = route_v.at[0]

    def chunk(k, acc):
        o = k * _LANES
        return acc + rowr[pl.ds(o, _LANES)] * rtr[pl.ds(o, _LANES)]

    acc = lax.fori_loop(0, _D // _LANES, chunk,
                        jnp.zeros((_LANES,), jnp.float32))
    logit = _lane_sum(acc)

    lv[0, 0] = jnp.full((_LANES,), logit, jnp.float32)
    pltpu.sync_copy(lv, out_hbm.at[pl.ds(b, 1), pl.ds(e, 1)])


_sc_logits = functools.partial(
    pl.kernel,
    out_type=jax.ShapeDtypeStruct((_B, _E, _LANES), jnp.float32),
    mesh=plsc.VectorSubcoreMesh(core_axis_name="c", subcore_axis_name="s",
                                num_cores=2, num_subcores=16),
    scratch_types=[
        pltpu.VMEM((_LANES,), jnp.int32),
        pltpu.VMEM((_LANES, _D), jnp.float32),
        pltpu.VMEM((1, _D), jnp.float32),
        pltpu.VMEM((1, 1, _LANES), jnp.float32),
    ],
)(_sc_logits_body)


def _tc_shared_body(x_ref, a_ref, sh_ref):
    xt = x_ref[0].astype(jnp.bfloat16)
    sh_ref[0] = lax.dot_general(
        xt, a_ref[...], (((1,), (1,)), ((), ())),
        preferred_element_type=jnp.float32).astype(jnp.bfloat16)


def _tc_out_body(lt_ref, sh_ref, bs_ref, o_ref, cb_ref):
    b = pl.program_id(0)
    s = pl.program_id(1)

    @pl.when(s == 0)
    def _routing():
        iota = lax.broadcasted_iota(jnp.int32, (_B, _E, _LANES), 0)
        lrow = jnp.sum(jnp.where(iota == b, lt_ref[...], 0.0),
                       axis=0)
        logits = [lrow[e, 0] for e in range(_E)]
        m1 = logits[0]
        i1 = jnp.int32(0)
        for e in range(1, _E):
            gt = logits[e] > m1
            i1 = jnp.where(gt, jnp.int32(e), i1)
            m1 = jnp.where(gt, logits[e], m1)
        m2 = jnp.float32(_NEG)
        i2 = jnp.int32(0)
        for e in range(_E):
            gt = (logits[e] > m2) & (i1 != e)
            i2 = jnp.where(gt, jnp.int32(e), i2)
            m2 = jnp.where(gt, logits[e], m2)
        e2 = jnp.exp(m2 - m1)
        denom = 1.0 + e2
        g1 = (_SCALING / denom)
        g2 = (_SCALING * e2 / denom)
        cb = jnp.zeros((_D, _R), dtype=jnp.float32)
        for e in range(_E):
            w_e = jnp.where(i1 == e, g1, 0.0) + jnp.where(i2 == e, g2, 0.0)
            cb = cb + w_e * bs_ref[e]
        cb_ref[...] = cb.astype(jnp.bfloat16)

    out = lax.dot_general(
        sh_ref[0], cb_ref[...], (((1,), (1,)), ((), ())),
        preferred_element_type=jnp.float32)
    o_ref[0] = out


@jax.jit
def kernel(x, eof_index, lora_A, lora_route, lora_Bs):
    eof16 = jnp.concatenate(
        [eof_index, jnp.zeros((_LANES - _B,), jnp.int32)])
    logits_tab = _sc_logits(x.reshape(_B * _S, _D), eof16, lora_route)
    grid = (_B, _S // _S_TILE)
    shared = pl.pallas_call(
        _tc_shared_body,
        grid=grid,
        in_specs=[
            pl.BlockSpec((1, _S_TILE, _D), lambda b, s: (b, s, 0)),
            pl.BlockSpec((_R, _D), lambda b, s: (0, 0)),
        ],
        out_specs=pl.BlockSpec((1, _S_TILE, _R), lambda b, s: (b, s, 0)),
        out_shape=jax.ShapeDtypeStruct((_B, _S, _R), jnp.bfloat16),
    )(x, lora_A.astype(jnp.bfloat16))
    return pl.pallas_call(
        _tc_out_body,
        grid=grid,
        in_specs=[
            pl.BlockSpec((_B, _E, _LANES), lambda b, s: (0, 0, 0)),
            pl.BlockSpec((1, _S_TILE, _R), lambda b, s: (b, s, 0)),
            pl.BlockSpec((_E, _D, _R), lambda b, s: (0, 0, 0)),
        ],
        out_specs=pl.BlockSpec((1, _S_TILE, _D), lambda b, s: (b, s, 0)),
        scratch_shapes=[pltpu.VMEM((_D, _R), jnp.bfloat16)],
        out_shape=jax.ShapeDtypeStruct((_B, _S, _D), jnp.float32),
    )(logits_tab, shared, lora_Bs)
